# baseline (speedup 1.0000x reference)
import functools

import numpy as np
import jax
import jax.numpy as jnp
from jax import lax
from jax.experimental import pallas as pl
from jax.experimental.pallas import tpu as pltpu
from jax.experimental.pallas import tpu_sc as plsc

_NUM_CORES = 2
_NUM_SUBCORES = 16
_NUM_WORKERS = _NUM_CORES * _NUM_SUBCORES

_LANES = 16
_C = 8
_NBUF = 4
_DEPTH = 2


def _positional_encoding(nb_words, nb_dimensions):
    X = np.arange(0, nb_words)
    Y = np.arange(0, nb_dimensions)
    Y, X = np.meshgrid(Y, X)
    TEMP = 10000
    temp1 = np.sin(X / np.power(TEMP, 2 * Y / nb_dimensions))
    temp2 = np.cos(X / np.power(TEMP, 2 * Y / nb_dimensions))
    Z = np.zeros((nb_words, nb_dimensions))
    Z[:, 0::2] = temp1[:, 0::2]
    Z[:, 1::2] = temp2[:, 1::2]
    return jnp.asarray(Z, dtype=jnp.float32)


def kernel(X, table):
    nb_texts, nb_tokens = X.shape
    d = table.shape[1]
    pe = _positional_encoding(nb_tokens, d)

    pe = jnp.concatenate([pe, pe[:_C]], axis=0)

    B = nb_texts * nb_tokens
    idx = X.reshape(B).astype(jnp.int32)
    b_per_w = B // _NUM_WORKERS
    nchunks = b_per_w // _C
    groups = d // _LANES

    mesh = plsc.VectorSubcoreMesh(core_axis_name="c", subcore_axis_name="s")

    @functools.partial(
        pl.kernel,
        out_type=jax.ShapeDtypeStruct((B, d), jnp.float32),
        mesh=mesh,
        scratch_types=[
            pltpu.VMEM((b_per_w,), jnp.int32),
            pltpu.VMEM((nb_tokens + _C, d), jnp.float32),
            [pltpu.VMEM((_C, d), jnp.float32)] * _NBUF,
            [pltpu.SemaphoreType.DMA] * _NBUF,
            [pltpu.SemaphoreType.DMA] * _NBUF,
        ],
    )
    def emb(table_h, idx_h, pe_h, out_h, idx_v, pe_v, bufs, in_sems, out_sems):
        wid = lax.axis_index("s") * _NUM_CORES + lax.axis_index("c")
        base = wid * b_per_w
        pltpu.sync_copy(idx_h.at[pl.ds(base, b_per_w)], idx_v)
        pltpu.sync_copy(pe_h, pe_v)

        def start_gather(c, b):
            pltpu.async_copy(
                table_h.at[idx_v.at[pl.ds(c * _C, _C)]], bufs[b], in_sems[b]
            )

        def wait_gather(b):
            pltpu.make_async_copy(
                table_h.at[idx_v.at[pl.ds(0, _C)]], bufs[b], in_sems[b]
            ).wait()

        def start_out(c, b):
            pltpu.async_copy(
                bufs[b], out_h.at[pl.ds(base + c * _C, _C)], out_sems[b]
            )

        def wait_out(b):
            pltpu.make_async_copy(
                bufs[b], out_h.at[pl.ds(base, _C)], out_sems[b]
            ).wait()

        def accumulate_pe(c, b):
            buf = bufs[b]
            pos0 = lax.rem(c * _C, nb_tokens)

            @plsc.parallel_loop(0, _C * groups)
            def _(i):
                r = lax.shift_right_logical(i, 5)
                g = lax.bitwise_and(i, groups - 1)
                sl = pl.ds(g * _LANES, _LANES)
                plsc.addupdate(buf.at[r, sl], pe_v[pos0 + r, sl])

        for k in range(_DEPTH):
            start_gather(k, k)

        @pl.loop(0, nchunks, step=_NBUF)
        def _(c):
            for j in range(_NBUF):
                k = c + j
                bn = (j + _DEPTH) % _NBUF

                @pl.when(k + _DEPTH < nchunks)
                def _():
                    @pl.when(k >= _NBUF - _DEPTH)
                    def _():
                        wait_out(bn)

                    start_gather(k + _DEPTH, bn)

                wait_gather(j)
                accumulate_pe(k, j)
                start_out(k, j)

        for b in range(_NBUF):
            wait_out(b)

    out = emb(table, idx, pe)
    return out.reshape(nb_texts, nb_tokens, d)


# speedup vs baseline: 2.0198x; 2.0198x over previous
import functools

import numpy as np
import jax
import jax.numpy as jnp
from jax import lax
from jax.experimental import pallas as pl
from jax.experimental.pallas import tpu as pltpu
from jax.experimental.pallas import tpu_sc as plsc

_NUM_CORES = 2
_NUM_SUBCORES = 16
_NUM_WORKERS = _NUM_CORES * _NUM_SUBCORES

_LANES = 16
_C = 8
_NBUF = 4
_DEPTH = 2


def _positional_encoding(nb_words, nb_dimensions):
    X = np.arange(0, nb_words)
    Y = np.arange(0, nb_dimensions)
    Y, X = np.meshgrid(Y, X)
    TEMP = 10000
    temp1 = np.sin(X / np.power(TEMP, 2 * Y / nb_dimensions))
    temp2 = np.cos(X / np.power(TEMP, 2 * Y / nb_dimensions))
    Z = np.zeros((nb_words, nb_dimensions))
    Z[:, 0::2] = temp1[:, 0::2]
    Z[:, 1::2] = temp2[:, 1::2]
    return jnp.asarray(Z, dtype=jnp.float32)


def kernel(X, table):
    nb_texts, nb_tokens = X.shape
    d = table.shape[1]
    pe = _positional_encoding(nb_tokens, d)

    pe = jnp.concatenate([pe, pe[:_C]], axis=0)

    B = nb_texts * nb_tokens
    idx = X.reshape(B).astype(jnp.int32)
    b_per_w = B // _NUM_WORKERS
    nchunks = b_per_w // _C
    groups = d // _LANES

    mesh = plsc.VectorSubcoreMesh(core_axis_name="c", subcore_axis_name="s")

    @functools.partial(
        pl.kernel,
        out_type=jax.ShapeDtypeStruct((B, d), jnp.float32),
        mesh=mesh,
        scratch_types=[
            pltpu.VMEM((b_per_w,), jnp.int32),
            pltpu.VMEM((nb_tokens + _C, d), jnp.float32),
            [pltpu.VMEM((_C, d), jnp.float32)] * _NBUF,
            [pltpu.SemaphoreType.DMA] * _NBUF,
            [pltpu.SemaphoreType.DMA] * _NBUF,
        ],
    )
    def emb(table_h, idx_h, pe_h, out_h, idx_v, pe_v, bufs, in_sems, out_sems):
        wid = lax.axis_index("s") * _NUM_CORES + lax.axis_index("c")
        base = wid * b_per_w
        pltpu.sync_copy(idx_h.at[pl.ds(base, b_per_w)], idx_v)
        pltpu.sync_copy(pe_h, pe_v)

        def start_gather(c, b):
            pltpu.async_copy(
                table_h.at[idx_v.at[pl.ds(c * _C, _C)]], bufs[b], in_sems[b]
            )

        def wait_gather(b):
            pltpu.make_async_copy(
                table_h.at[idx_v.at[pl.ds(0, _C)]], bufs[b], in_sems[b]
            ).wait()

        def start_out(c, b):
            pltpu.async_copy(
                bufs[b], out_h.at[pl.ds(base + c * _C, _C)], out_sems[b]
            )

        def wait_out(b):
            pltpu.make_async_copy(
                bufs[b], out_h.at[pl.ds(base, _C)], out_sems[b]
            ).wait()

        def accumulate_pe(c, b):
            buf = bufs[b]
            pos0 = lax.rem(c * _C, nb_tokens)

            @plsc.parallel_loop(0, 2 * _C)
            def _(i):
                r = lax.shift_right_logical(i, 1)
                h = lax.bitwise_and(i, 1) * (groups // 2 * _LANES)
                pos = pos0 + r
                for g in range(groups // 2):
                    sl = pl.ds(h + g * _LANES, _LANES)
                    plsc.addupdate(buf.at[r, sl], pe_v[pos, sl])

        for k in range(_DEPTH):
            start_gather(k, k)

        @pl.loop(0, nchunks, step=_NBUF)
        def _(c):
            for j in range(_NBUF):
                k = c + j
                bn = (j + _DEPTH) % _NBUF

                @pl.when(k + _DEPTH < nchunks)
                def _():
                    @pl.when(k >= _NBUF - _DEPTH)
                    def _():
                        wait_out(bn)

                    start_gather(k + _DEPTH, bn)

                wait_gather(j)
                accumulate_pe(k, j)
                start_out(k, j)

        for b in range(_NBUF):
            wait_out(b)

    out = emb(table, idx, pe)
    return out.reshape(nb_texts, nb_tokens, d)


# speedup vs baseline: 2.0399x; 1.0100x over previous
import functools

import numpy as np
import jax
import jax.numpy as jnp
from jax import lax
from jax.experimental import pallas as pl
from jax.experimental.pallas import tpu as pltpu
from jax.experimental.pallas import tpu_sc as plsc

_NUM_CORES = 2
_NUM_SUBCORES = 16
_NUM_WORKERS = _NUM_CORES * _NUM_SUBCORES

_LANES = 16
_C = 8
_NBUF = 4
_DEPTH = 2


def _positional_encoding(nb_words, nb_dimensions):
    X = np.arange(0, nb_words)
    Y = np.arange(0, nb_dimensions)
    Y, X = np.meshgrid(Y, X)
    TEMP = 10000
    temp1 = np.sin(X / np.power(TEMP, 2 * Y / nb_dimensions))
    temp2 = np.cos(X / np.power(TEMP, 2 * Y / nb_dimensions))
    Z = np.zeros((nb_words, nb_dimensions))
    Z[:, 0::2] = temp1[:, 0::2]
    Z[:, 1::2] = temp2[:, 1::2]
    return jnp.asarray(Z, dtype=jnp.float32)


def kernel(X, table):
    nb_texts, nb_tokens = X.shape
    d = table.shape[1]
    pe = _positional_encoding(nb_tokens, d)

    pe = jnp.concatenate([pe, pe[:_C]], axis=0)

    B = nb_texts * nb_tokens
    idx = X.reshape(B).astype(jnp.int32)
    b_per_w = B // _NUM_WORKERS
    nchunks = b_per_w // _C
    groups = d // _LANES

    mesh = plsc.VectorSubcoreMesh(core_axis_name="c", subcore_axis_name="s")

    @functools.partial(
        pl.kernel,
        out_type=jax.ShapeDtypeStruct((B, d), jnp.float32),
        mesh=mesh,
        scratch_types=[
            pltpu.VMEM((b_per_w,), jnp.int32),
            pltpu.VMEM((nb_tokens + _C, d), jnp.float32),
            [pltpu.VMEM((_C, d), jnp.float32)] * _NBUF,
            [pltpu.SemaphoreType.DMA] * _NBUF,
            [pltpu.SemaphoreType.DMA] * _NBUF,
        ],
    )
    def emb(table_h, idx_h, pe_h, out_h, idx_v, pe_v, bufs, in_sems, out_sems):
        wid = lax.axis_index("s") * _NUM_CORES + lax.axis_index("c")
        base = wid * b_per_w
        pltpu.sync_copy(idx_h.at[pl.ds(base, b_per_w)], idx_v)
        pltpu.sync_copy(pe_h, pe_v)

        def start_gather(c, b):
            pltpu.async_copy(
                table_h.at[idx_v.at[pl.ds(c * _C, _C)]], bufs[b], in_sems[b]
            )

        def wait_gather(b):
            pltpu.make_async_copy(
                table_h.at[idx_v.at[pl.ds(0, _C)]], bufs[b], in_sems[b]
            ).wait()

        def start_out(c, b):
            pltpu.async_copy(
                bufs[b], out_h.at[pl.ds(base + c * _C, _C)], out_sems[b]
            )

        def wait_out(b):
            pltpu.make_async_copy(
                bufs[b], out_h.at[pl.ds(base, _C)], out_sems[b]
            ).wait()

        def accumulate_pe(c, b):
            buf = bufs[b]
            pos0 = lax.rem(c * _C, nb_tokens)

            @plsc.parallel_loop(0, 4 * _C)
            def _(i):
                r = lax.shift_right_logical(i, 2)
                h = lax.bitwise_and(i, 3) * (groups // 4 * _LANES)
                pos = pos0 + r
                for g in range(groups // 4):
                    sl = pl.ds(h + g * _LANES, _LANES)
                    plsc.addupdate(buf.at[r, sl], pe_v[pos, sl])

        for k in range(_DEPTH):
            start_gather(k, k)

        @pl.loop(0, nchunks, step=_NBUF)
        def _(c):
            for j in range(_NBUF):
                k = c + j
                bn = (j + _DEPTH) % _NBUF

                @pl.when(k + _DEPTH < nchunks)
                def _():
                    @pl.when(k >= _NBUF - _DEPTH)
                    def _():
                        wait_out(bn)

                    start_gather(k + _DEPTH, bn)

                wait_gather(j)
                accumulate_pe(k, j)
                start_out(k, j)

        for b in range(_NBUF):
            wait_out(b)

    out = emb(table, idx, pe)
    return out.reshape(nb_texts, nb_tokens, d)
